# baseline (speedup 1.0000x reference)
import numpy as np
import jax
import jax.numpy as jnp
from jax import lax
from jax.experimental import pallas as pl
from jax.experimental.pallas import tpu as pltpu
from jax.experimental.pallas import tpu_sc as plsc

_FD = 3846
_F = 26
_D = 16
_B = 4096
_V = _F * _FD
_P = _F * (_F - 1) // 2

_NW = 32
_L = 16
_DQ = 8
_UPP = _D // _DQ
_NU = _P * _UPP
_BW = 4096
_VPAD = ((_V + 127) >> 7) << 7
_CMAX = _VPAD - _BW
_NG = _B // _L


_CH = 2048
_NCH = _B // _CH
_GPC = _CH // _L


def _worker(wid, tabT, xT, out, bandA, bandB, xcolA, xcolB, stage0, stage1,
            sem_a, sem_b, sem_x, sem_o):
    lo = (wid * _NU) >> 5
    hi = ((wid + 1) * _NU) >> 5

    p0 = lo >> 1
    dq0 = lo & 1

    def scan_i(t, acc):
        return acc + jnp.where((t * (51 - t)) // 2 <= p0, 1, 0)

    i0 = lax.fori_loop(1, _F, scan_i, jnp.int32(0))
    j0 = p0 - (i0 * (51 - i0)) // 2 + i0 + 1

    def unit_step(u, carry):
        i, j, dq = carry
        p = u >> 1
        offA = i * _FD
        offB = j * _FD
        cA = jnp.minimum((offA >> 7) << 7, _CMAX)
        cB = jnp.minimum((offB >> 7) << 7, _CMAX)
        shA = offA - cA
        shB = offB - cB
        cA = pl.multiple_of(cA, 128)
        cB = pl.multiple_of(cB, 128)
        rA = pl.multiple_of(j * _D + dq * _DQ, 8)
        rB = pl.multiple_of(i * _D + dq * _DQ, 8)

        cpa = pltpu.async_copy(tabT.at[pl.ds(rA, _DQ), pl.ds(cA, _BW)],
                               bandA, sem_a)
        cpb = pltpu.async_copy(tabT.at[pl.ds(rB, _DQ), pl.ds(cB, _BW)],
                               bandB, sem_b)
        cpx1 = pltpu.async_copy(xT.at[i], xcolA, sem_x)
        cpx2 = pltpu.async_copy(xT.at[j], xcolB, sem_x)
        cpa.wait()
        cpb.wait()
        cpx1.wait()
        cpx2.wait()

        dslice = pl.ds(dq * _DQ, _DQ)
        for c in range(_NCH):
            stage = (stage0, stage1)[c & 1]
            if c >= 2:
                pltpu.make_async_copy(
                    stage, out.at[p, dslice, pl.ds(c * _CH, _CH)],
                    sem_o).wait()
            else:
                @pl.when(u > lo)
                def _():
                    pltpu.make_async_copy(
                        stage, out.at[p, dslice, pl.ds(c * _CH, _CH)],
                        sem_o).wait()

            @plsc.parallel_loop(0, _GPC, 1, unroll=8)
            def group(g):
                gsl = pl.ds(g * _L, _L)
                xsl = pl.ds(c * _CH + g * _L, _L)
                ia = xcolA[xsl] + shA
                ib = xcolB[xsl] + shB
                for d in range(_DQ):
                    dv = jnp.full((_L,), d, jnp.int32)
                    a = plsc.load_gather(bandA, [dv, ia])
                    b = plsc.load_gather(bandB, [dv, ib])
                    stage[d, gsl] = a * b
            pltpu.async_copy(stage, out.at[p, dslice, pl.ds(c * _CH, _CH)],
                             sem_o)

        dq2 = dq + 1
        wd = dq2 >= _UPP
        dq2 = jnp.where(wd, 0, dq2)
        j2 = jnp.where(wd, j + 1, j)
        wj = j2 >= _F
        i2 = jnp.where(wj, i + 1, i)
        j2 = jnp.where(wj, i + 2, j2)
        return i2, j2, dq2

    lax.fori_loop(lo, hi, unit_step, (i0, j0, jnp.int32(dq0)))

    for s in (stage0, stage1):
        pltpu.make_async_copy(
            s, out.at[0, pl.ds(0, _DQ), pl.ds(0, _CH)], sem_o).wait()


def _ffm_body(tabT, xT, out, *rest):
    wid = lax.axis_index("s") * 2 + lax.axis_index("c")
    _worker(wid, tabT, xT, out, *rest)


def kernel(x, tables):
    tabT = tables.transpose(0, 2, 1).reshape(_F * _D, _V)
    xT = x.T
    mesh = plsc.VectorSubcoreMesh(core_axis_name="c", subcore_axis_name="s")
    run = pl.kernel(
        _ffm_body,
        out_type=jax.ShapeDtypeStruct((_P, _D, _B), jnp.float32),
        mesh=mesh,
        compiler_params=pltpu.CompilerParams(needs_layout_passes=False),
        scratch_types=[
            pltpu.VMEM((_DQ, _BW), jnp.float32),
            pltpu.VMEM((_DQ, _BW), jnp.float32),
            pltpu.VMEM((_B,), jnp.int32),
            pltpu.VMEM((_B,), jnp.int32),
            pltpu.VMEM((_DQ, _CH), jnp.float32),
            pltpu.VMEM((_DQ, _CH), jnp.float32),
            pltpu.SemaphoreType.DMA,
            pltpu.SemaphoreType.DMA,
            pltpu.SemaphoreType.DMA,
            pltpu.SemaphoreType.DMA,
        ],
    )
    out = run(tabT, xT)
    return out.transpose(2, 0, 1)


# speedup vs baseline: 1.0768x; 1.0768x over previous
import numpy as np
import jax
import jax.numpy as jnp
from jax import lax
from jax.experimental import pallas as pl
from jax.experimental.pallas import tpu as pltpu
from jax.experimental.pallas import tpu_sc as plsc

_FD = 3846
_F = 26
_D = 16
_B = 4096
_V = _F * _FD
_P = _F * (_F - 1) // 2

_NW = 32
_L = 16
_DQ = 8
_UPP = _D // _DQ
_NU = _P * _UPP
_BW = 4096
_VPAD = ((_V + 127) >> 7) << 7
_CMAX = _VPAD - _BW
_NG = _B // _L


_CH = 2048
_NCH = _B // _CH
_GPC = _CH // _L


def _worker(wid, tabT, xT, out, bandA, bandB, xcolA, xcolB, stage0, stage1,
            sem_a, sem_b, sem_x, sem_o):
    lo = (wid * _NU) >> 5
    hi = ((wid + 1) * _NU) >> 5

    p0 = lo >> 1
    dq0 = lo & 1

    def scan_i(t, acc):
        return acc + jnp.where((t * (51 - t)) // 2 <= p0, 1, 0)

    i0 = lax.fori_loop(1, _F, scan_i, jnp.int32(0))
    j0 = p0 - (i0 * (51 - i0)) // 2 + i0 + 1

    def unit_step(u, carry):
        i, j, dq = carry
        p = u >> 1
        offA = i * _FD
        offB = j * _FD
        cA = jnp.minimum((offA >> 7) << 7, _CMAX)
        cB = jnp.minimum((offB >> 7) << 7, _CMAX)
        shA = offA - cA
        shB = offB - cB
        cA = pl.multiple_of(cA, 128)
        cB = pl.multiple_of(cB, 128)
        rA = pl.multiple_of(j * _D + dq * _DQ, 8)
        rB = pl.multiple_of(i * _D + dq * _DQ, 8)

        cpa = pltpu.async_copy(tabT.at[pl.ds(rA, _DQ), pl.ds(cA, _BW)],
                               bandA, sem_a)
        cpb = pltpu.async_copy(tabT.at[pl.ds(rB, _DQ), pl.ds(cB, _BW)],
                               bandB, sem_b)
        cpx1 = pltpu.async_copy(xT.at[i], xcolA, sem_x)
        cpx2 = pltpu.async_copy(xT.at[j], xcolB, sem_x)
        cpa.wait()
        cpb.wait()
        cpx1.wait()
        cpx2.wait()

        dslice = pl.ds(dq * _DQ, _DQ)
        for c in range(_NCH):
            stage = (stage0, stage1)[c & 1]
            if c >= 2:
                pltpu.make_async_copy(
                    stage, out.at[p, dslice, pl.ds(c * _CH, _CH)],
                    sem_o).wait()
            else:
                @pl.when(u > lo)
                def _():
                    pltpu.make_async_copy(
                        stage, out.at[p, dslice, pl.ds(c * _CH, _CH)],
                        sem_o).wait()

            @plsc.parallel_loop(0, _GPC, 1, unroll=4)
            def group(g):
                gsl = pl.ds(g * _L, _L)
                xsl = pl.ds(c * _CH + g * _L, _L)
                ia = xcolA[xsl] + shA
                ib = xcolB[xsl] + shB
                for d in range(_DQ):
                    dv = jnp.full((_L,), d, jnp.int32)
                    a = plsc.load_gather(bandA, [dv, ia])
                    b = plsc.load_gather(bandB, [dv, ib])
                    stage[d, gsl] = a * b
            pltpu.async_copy(stage, out.at[p, dslice, pl.ds(c * _CH, _CH)],
                             sem_o)

        dq2 = dq + 1
        wd = dq2 >= _UPP
        dq2 = jnp.where(wd, 0, dq2)
        j2 = jnp.where(wd, j + 1, j)
        wj = j2 >= _F
        i2 = jnp.where(wj, i + 1, i)
        j2 = jnp.where(wj, i + 2, j2)
        return i2, j2, dq2

    lax.fori_loop(lo, hi, unit_step, (i0, j0, jnp.int32(dq0)))

    for s in (stage0, stage1):
        pltpu.make_async_copy(
            s, out.at[0, pl.ds(0, _DQ), pl.ds(0, _CH)], sem_o).wait()


def _ffm_body(tabT, xT, out, *rest):
    wid = lax.axis_index("s") * 2 + lax.axis_index("c")
    _worker(wid, tabT, xT, out, *rest)


def kernel(x, tables):
    tabT = tables.transpose(0, 2, 1).reshape(_F * _D, _V)
    xT = x.T
    mesh = plsc.VectorSubcoreMesh(core_axis_name="c", subcore_axis_name="s")
    run = pl.kernel(
        _ffm_body,
        out_type=jax.ShapeDtypeStruct((_P, _D, _B), jnp.float32),
        mesh=mesh,
        compiler_params=pltpu.CompilerParams(needs_layout_passes=False),
        scratch_types=[
            pltpu.VMEM((_DQ, _BW), jnp.float32),
            pltpu.VMEM((_DQ, _BW), jnp.float32),
            pltpu.VMEM((_B,), jnp.int32),
            pltpu.VMEM((_B,), jnp.int32),
            pltpu.VMEM((_DQ, _CH), jnp.float32),
            pltpu.VMEM((_DQ, _CH), jnp.float32),
            pltpu.SemaphoreType.DMA,
            pltpu.SemaphoreType.DMA,
            pltpu.SemaphoreType.DMA,
            pltpu.SemaphoreType.DMA,
        ],
    )
    out = run(tabT, xT)
    return out.transpose(2, 0, 1)


# speedup vs baseline: 1.0992x; 1.0208x over previous
import numpy as np
import jax
import jax.numpy as jnp
from jax import lax
from jax.experimental import pallas as pl
from jax.experimental.pallas import tpu as pltpu
from jax.experimental.pallas import tpu_sc as plsc

_FD = 3846
_F = 26
_D = 16
_B = 4096
_V = _F * _FD
_P = _F * (_F - 1) // 2

_NW = 32
_L = 16
_DQ = 8
_UPP = _D // _DQ
_NU = _P * _UPP
_BW = 4096
_VPAD = ((_V + 127) >> 7) << 7
_CMAX = _VPAD - _BW
_NG = _B // _L


_CH = 2048
_NCH = _B // _CH
_GPC = _CH // _L


def _worker(wid, tabT, xT, out, bandA, bandB, xcolA, xcolB, stage0, stage1,
            sem_a, sem_b, sem_x, sem_o):
    lo = (wid * _NU) >> 5
    hi = ((wid + 1) * _NU) >> 5

    p0 = lo >> 1
    dq0 = lo & 1

    def scan_i(t, acc):
        return acc + jnp.where((t * (51 - t)) // 2 <= p0, 1, 0)

    i0 = lax.fori_loop(1, _F, scan_i, jnp.int32(0))
    j0 = p0 - (i0 * (51 - i0)) // 2 + i0 + 1

    def unit_step(u, carry):
        i, j, dq = carry
        p = u >> 1
        offA = i * _FD
        offB = j * _FD
        cA = jnp.minimum((offA >> 7) << 7, _CMAX)
        cB = jnp.minimum((offB >> 7) << 7, _CMAX)
        shA = offA - cA
        shB = offB - cB
        cA = pl.multiple_of(cA, 128)
        cB = pl.multiple_of(cB, 128)
        rA = pl.multiple_of(j * _D + dq * _DQ, 8)
        rB = pl.multiple_of(i * _D + dq * _DQ, 8)

        cpa = pltpu.async_copy(tabT.at[pl.ds(rA, _DQ), pl.ds(cA, _BW)],
                               bandA, sem_a)
        cpb = pltpu.async_copy(tabT.at[pl.ds(rB, _DQ), pl.ds(cB, _BW)],
                               bandB, sem_b)
        @pl.when((dq == 0) | (u == lo))
        def _():
            cpx1 = pltpu.async_copy(xT.at[i], xcolA, sem_x)
            cpx2 = pltpu.async_copy(xT.at[j], xcolB, sem_x)
            cpx1.wait()
            cpx2.wait()

        cpa.wait()
        cpb.wait()

        dslice = pl.ds(dq * _DQ, _DQ)
        for c in range(_NCH):
            stage = (stage0, stage1)[c & 1]
            if c >= 2:
                pltpu.make_async_copy(
                    stage, out.at[p, dslice, pl.ds(c * _CH, _CH)],
                    sem_o).wait()
            else:
                @pl.when(u > lo)
                def _():
                    pltpu.make_async_copy(
                        stage, out.at[p, dslice, pl.ds(c * _CH, _CH)],
                        sem_o).wait()

            @plsc.parallel_loop(0, _GPC, 1, unroll=4)
            def group(g):
                gsl = pl.ds(g * _L, _L)
                xsl = pl.ds(c * _CH + g * _L, _L)
                ia = xcolA[xsl] + shA
                ib = xcolB[xsl] + shB
                for d in range(_DQ):
                    dv = jnp.full((_L,), d, jnp.int32)
                    a = plsc.load_gather(bandA, [dv, ia])
                    b = plsc.load_gather(bandB, [dv, ib])
                    stage[d, gsl] = a * b
            pltpu.async_copy(stage, out.at[p, dslice, pl.ds(c * _CH, _CH)],
                             sem_o)

        dq2 = dq + 1
        wd = dq2 >= _UPP
        dq2 = jnp.where(wd, 0, dq2)
        j2 = jnp.where(wd, j + 1, j)
        wj = j2 >= _F
        i2 = jnp.where(wj, i + 1, i)
        j2 = jnp.where(wj, i + 2, j2)
        return i2, j2, dq2

    lax.fori_loop(lo, hi, unit_step, (i0, j0, jnp.int32(dq0)))

    for s in (stage0, stage1):
        pltpu.make_async_copy(
            s, out.at[0, pl.ds(0, _DQ), pl.ds(0, _CH)], sem_o).wait()


def _ffm_body(tabT, xT, out, *rest):
    wid = lax.axis_index("s") * 2 + lax.axis_index("c")
    _worker(wid, tabT, xT, out, *rest)


def kernel(x, tables):
    tabT = tables.transpose(0, 2, 1).reshape(_F * _D, _V)
    xT = x.T
    mesh = plsc.VectorSubcoreMesh(core_axis_name="c", subcore_axis_name="s")
    run = pl.kernel(
        _ffm_body,
        out_type=jax.ShapeDtypeStruct((_P, _D, _B), jnp.float32),
        mesh=mesh,
        compiler_params=pltpu.CompilerParams(needs_layout_passes=False),
        scratch_types=[
            pltpu.VMEM((_DQ, _BW), jnp.float32),
            pltpu.VMEM((_DQ, _BW), jnp.float32),
            pltpu.VMEM((_B,), jnp.int32),
            pltpu.VMEM((_B,), jnp.int32),
            pltpu.VMEM((_DQ, _CH), jnp.float32),
            pltpu.VMEM((_DQ, _CH), jnp.float32),
            pltpu.SemaphoreType.DMA,
            pltpu.SemaphoreType.DMA,
            pltpu.SemaphoreType.DMA,
            pltpu.SemaphoreType.DMA,
        ],
    )
    out = run(tabT, xT)
    return out.transpose(2, 0, 1)
